# Initial kernel scaffold; baseline (speedup 1.0000x reference)
import jax
import jax.numpy as jnp
from jax.experimental import pallas as pl


def kernel(distances_uv, vectors_uv, atomic_charges, atomic_dipoles, idx_u, idx_v):
    raise NotImplementedError("write your pallas kernel here")


# speedup vs baseline: 16.0564x; 16.0564x over previous
import functools

import jax
import jax.numpy as jnp
from jax import lax
from jax.experimental import pallas as pl
from jax.experimental.pallas import tpu as pltpu
from jax.experimental.pallas import tpu_sc as plsc

CUTOFF = 10.0
CUTOFF_SR = 2.0
KEHALF = 7.199822675975274
N_NODES = 100000
N_EDGES = 6400000

NC = 2
NS = 16
L = 16
NW = NC * NS
CH = 2000
EPW = N_EDGES // NW
NCHUNK = EPW // CH
G = CH // L

_P0 = 0.9999991252
_P1 = -2.467364153
_P2 = 2.028983924
_P3 = -0.6661269207
_P4 = 0.1147943105
_P5 = -0.01028713516

_CHI1 = 1.0 / CUTOFF
_CHI2 = _CHI1 * _CHI1
_CHI3 = _CHI2 * _CHI1


def _sc_body(d_hbm, vec_hbm, chg_hbm, dpx_hbm, dpy_hbm, dpz_hbm,
             iu_hbm, iv_hbm, out_hbm,
             iu_v, iv_v, d_v, vec_v,
             qu_v, qv_v, dux_v, duy_v, duz_v, dvx_v, dvy_v, dvz_v,
             out_v, sem):
    wid = lax.axis_index("s") * NC + lax.axis_index("c")
    base0 = wid * EPW
    lanes = lax.iota(jnp.int32, L)

    def chunk_body(j, carry):
        base = base0 + j * CH
        pltpu.sync_copy(iu_hbm.at[pl.ds(base, CH)], iu_v)
        pltpu.sync_copy(iv_hbm.at[pl.ds(base, CH)], iv_v)
        cps = [
            pltpu.async_copy(chg_hbm.at[iu_v], qu_v, sem),
            pltpu.async_copy(chg_hbm.at[iv_v], qv_v, sem),
            pltpu.async_copy(dpx_hbm.at[iu_v], dux_v, sem),
            pltpu.async_copy(dpy_hbm.at[iu_v], duy_v, sem),
            pltpu.async_copy(dpz_hbm.at[iu_v], duz_v, sem),
            pltpu.async_copy(dpx_hbm.at[iv_v], dvx_v, sem),
            pltpu.async_copy(dpy_hbm.at[iv_v], dvy_v, sem),
            pltpu.async_copy(dpz_hbm.at[iv_v], dvz_v, sem),
        ]
        pltpu.sync_copy(d_hbm.at[pl.ds(base, CH)], d_v)
        pltpu.sync_copy(vec_hbm.at[pl.ds(3 * base, 3 * CH)], vec_v)
        for c in cps:
            c.wait()

        def group(i, c):
            s = i * L
            d = d_v[pl.ds(s, L)]
            d2 = d * d
            inv_d = 1.0 / d
            a = d2 + 1.0
            yi = 0x5F3759DF - (lax.bitcast_convert_type(a, jnp.int32) >> 1)
            y = lax.bitcast_convert_type(yi, jnp.float32)
            xh = a * 0.5
            y = y * (1.5 - xh * y * y)
            inv_dd = y * (1.5 - xh * y * y)
            t = d2 * 0.25
            sw = _P5
            sw = sw * t + _P4
            sw = sw * t + _P3
            sw = sw * t + _P2
            sw = sw * t + _P1
            sw = sw * t + _P0
            sw = jnp.where(d < CUTOFF_SR, sw, 0.0)
            chi = inv_d + sw * (inv_dd - inv_d)
            chi2 = chi * chi
            chi3 = chi2 * chi
            qu = qu_v[pl.ds(s, L)]
            qv = qv_v[pl.ds(s, L)]
            dux = dux_v[pl.ds(s, L)]
            duy = duy_v[pl.ds(s, L)]
            duz = duz_v[pl.ds(s, L)]
            dvx = dvx_v[pl.ds(s, L)]
            dvy = dvy_v[pl.ds(s, L)]
            dvz = dvz_v[pl.ds(s, L)]
            i3 = 3 * s + 3 * lanes
            vx = plsc.load_gather(vec_v, [i3])
            vy = plsc.load_gather(vec_v, [i3 + 1])
            vz = plsc.load_gather(vec_v, [i3 + 2])
            dot_uv = (vx * dvx + vy * dvy + vz * dvz) * inv_d
            dot_vu = (vx * dux + vy * duy + vz * duz) * inv_d
            dudv = dux * dvx + duy * dvy + duz * dvz
            e = qu * qv * (chi - _CHI1)
            e = e + 2.0 * qu * dot_uv * (chi2 - _CHI2)
            e = e + (dudv - 3.0 * dot_uv * dot_vu) * (chi3 - _CHI3)
            e = KEHALF * e
            e = jnp.where(d <= CUTOFF, e, 0.0)
            out_v[pl.ds(s, L)] = e
            return c

        lax.fori_loop(0, G, group, 0, unroll=False)
        pltpu.sync_copy(out_v, out_hbm.at[pl.ds(base, CH)])
        return carry

    lax.fori_loop(0, NCHUNK, chunk_body, 0, unroll=False)


@jax.jit
def _run(distances, vec_flat, chg, dpx, dpy, dpz, idx_u, idx_v):
    mesh = plsc.VectorSubcoreMesh(core_axis_name="c", subcore_axis_name="s")
    f = pl.kernel(
        _sc_body,
        out_type=jax.ShapeDtypeStruct((N_EDGES,), jnp.float32),
        mesh=mesh,
        compiler_params=pltpu.CompilerParams(needs_layout_passes=False),
        scratch_types=[
            pltpu.VMEM((CH,), jnp.int32),
            pltpu.VMEM((CH,), jnp.int32),
            pltpu.VMEM((CH,), jnp.float32),
            pltpu.VMEM((3 * CH,), jnp.float32),
            pltpu.VMEM((CH,), jnp.float32),
            pltpu.VMEM((CH,), jnp.float32),
            pltpu.VMEM((CH,), jnp.float32),
            pltpu.VMEM((CH,), jnp.float32),
            pltpu.VMEM((CH,), jnp.float32),
            pltpu.VMEM((CH,), jnp.float32),
            pltpu.VMEM((CH,), jnp.float32),
            pltpu.VMEM((CH,), jnp.float32),
            pltpu.VMEM((CH,), jnp.float32),
            pltpu.SemaphoreType.DMA,
        ],
    )
    return f(distances, vec_flat, chg, dpx, dpy, dpz, idx_u, idx_v)


def kernel(distances_uv, vectors_uv, atomic_charges, atomic_dipoles, idx_u, idx_v):
    dpx = atomic_dipoles[:, 0]
    dpy = atomic_dipoles[:, 1]
    dpz = atomic_dipoles[:, 2]
    vec_flat = vectors_uv.reshape(-1)
    return _run(distances_uv, vec_flat, atomic_charges, dpx, dpy, dpz,
                idx_u.astype(jnp.int32), idx_v.astype(jnp.int32))
